# Initial kernel scaffold; baseline (speedup 1.0000x reference)
import jax
import jax.numpy as jnp
from jax.experimental import pallas as pl


def kernel(img_feat0, img_feat1, img_feat2, img_feat3, input, batch):
    raise NotImplementedError("write your pallas kernel here")


# speedup vs baseline: 2.8016x; 2.8016x over previous
import jax
import jax.numpy as jnp
from jax import lax
from jax.experimental import pallas as pl
from jax.experimental.pallas import tpu as pltpu
from jax.experimental.pallas import tpu_sc as plsc

N = 10000
CHUNK = 80
NUM_CHUNKS = N // CHUNK
NW = 32
LANES = 16
IMG_SIZES = (56, 28, 14, 7)
CHANNELS = (64, 128, 256, 512)
COL_OFF = (0, 64, 192, 448)
OUT_COLS = 960


def _body(t0, t1, t2, t3, in0, in1, in2, out,
          v0, v1, v2, i0, i1, i2, i3, r0, r1, r2, r3, sem):
    tabs = (t0, t1, t2, t3)
    idxs = (i0, i1, i2, i3)
    rows = (r0, r1, r2, r3)
    wid = lax.axis_index("s") * 2 + lax.axis_index("c")

    def chunk_body(c):
        base = c * CHUNK
        pltpu.sync_copy(in0.at[pl.ds(base, CHUNK)], v0)
        pltpu.sync_copy(in1.at[pl.ds(base, CHUNK)], v1)
        pltpu.sync_copy(in2.at[pl.ds(base, CHUNK)], v2)
        for i in range(CHUNK // LANES):
            sl = pl.ds(i * LANES, LANES)
            a0 = v0[sl]
            a1 = v1[sl]
            a2 = v2[sl]
            h = 248.0 * (a1 / a2) + 111.5
            w = 248.0 * (a0 / (-a2)) + 111.5
            h = jnp.minimum(jnp.maximum(h, 0.0), 223.0)
            w = jnp.minimum(jnp.maximum(w, 0.0), 223.0)
            for s, size in enumerate(IMG_SIZES):
                x = h * (size / 224.0)
                y = w * (size / 224.0)
                xi = x.astype(jnp.int32)
                yi = y.astype(jnp.int32)
                xi = jnp.minimum(jnp.maximum(xi, 0), size - 1)
                yi = jnp.minimum(jnp.maximum(yi, 0), size - 1)
                ok = ((x > xi.astype(jnp.float32))
                      & (y > yi.astype(jnp.float32))
                      & (xi < size - 1) & (yi < size - 1))
                idx = xi * size + yi
                idxs[s][sl] = jnp.where(ok, idx, size * size)
        cps = [pltpu.async_copy(tabs[s].at[idxs[s]], rows[s], sem)
               for s in range(4)]
        for cp in cps:
            cp.wait()
        for s in range(4):
            pltpu.sync_copy(
                rows[s],
                out.at[pl.ds(base, CHUNK), pl.ds(COL_OFF[s], CHANNELS[s])])

    for j in range((NUM_CHUNKS + NW - 1) // NW):
        c = wid + NW * j
        if (j + 1) * NW <= NUM_CHUNKS:
            chunk_body(c)
        else:
            @pl.when(c < NUM_CHUNKS)
            def _():
                chunk_body(c)


def kernel(img_feat0, img_feat1, img_feat2, img_feat3, input, batch):
    feats = (img_feat0, img_feat1, img_feat2, img_feat3)
    tables = []
    for f, size, ch in zip(feats, IMG_SIZES, CHANNELS):
        t = f[batch].reshape(ch, size * size).T
        t = jnp.concatenate([t, jnp.zeros((1, ch), jnp.float32)], axis=0)
        tables.append(t)
    in0 = input[:, 0]
    in1 = input[:, 1]
    in2 = input[:, 2]

    mesh = plsc.VectorSubcoreMesh(core_axis_name="c", subcore_axis_name="s")
    scratch = (
        [pltpu.VMEM((CHUNK,), jnp.float32) for _ in range(3)]
        + [pltpu.VMEM((CHUNK,), jnp.int32) for _ in range(4)]
        + [pltpu.VMEM((CHUNK, ch), jnp.float32) for ch in CHANNELS]
        + [pltpu.SemaphoreType.DMA]
    )
    run = pl.kernel(
        _body,
        out_type=jax.ShapeDtypeStruct((N, OUT_COLS), jnp.float32),
        mesh=mesh,
        scratch_types=scratch,
        compiler_params=pltpu.CompilerParams(use_tc_tiling_on_sc=False),
    )
    return run(*tables, in0, in1, in2)
